# baseline (speedup 1.0000x reference)
import functools

import jax
import jax.numpy as jnp
from jax import lax
from jax.experimental import pallas as pl
from jax.experimental.pallas import tpu as pltpu
from jax.experimental.pallas import tpu_sc as plsc

N = 10000
E = 320000
D = 128

NC = 2
NS = 16
NW = NC * NS
CHUNK = 128
NCHUNKS = 79
EDGES_PER_W = NCHUNKS * CHUNK
E_PAD = EDGES_PER_W * NW
N_PAD = 10240
ROWS_PER_TILE = N_PAD // NS
DEG_W = 128


def _sc_aggregate(h, packed3, zrows):
    mesh = plsc.VectorSubcoreMesh(core_axis_name="c", subcore_axis_name="s")

    out_type = jax.ShapeDtypeStruct((NC, N_PAD, D), jnp.float32)

    scratch = [
        pltpu.VMEM((NCHUNKS, CHUNK), jnp.int32),
        pltpu.VMEM((CHUNK,), jnp.int32),
        pltpu.VMEM((CHUNK,), jnp.int32),
        pltpu.VMEM((CHUNK,), jnp.int32),
        pltpu.VMEM((CHUNK,), jnp.int32),
        pltpu.VMEM((CHUNK, D), jnp.float32),
        pltpu.VMEM((CHUNK, D), jnp.float32),
        pltpu.VMEM_SHARED((N_PAD, D), jnp.float32),
        pltpu.SemaphoreType.DMA,
    ]

    def body(h_hbm, packed_hbm, z_hbm, *rest):
        (acc_out, packed_v, c0, c1, r0, r1, m0, m1, acc_sh, sem) = rest
        cid = lax.axis_index("c")
        sid = lax.axis_index("s")
        wid = sid * NC + cid

        pltpu.sync_copy(packed_hbm.at[wid], packed_v)

        base = sid * ROWS_PER_TILE
        pltpu.sync_copy(z_hbm, acc_sh.at[pl.ds(base, ROWS_PER_TILE)])
        plsc.subcore_barrier()

        def unpack(j, c_buf, r_buf):
            for k in range(CHUNK // 16):
                p = packed_v[j, pl.ds(k * 16, 16)]
                c_buf[pl.ds(k * 16, 16)] = jnp.bitwise_and(p, 0xFFFF)
                r_buf[pl.ds(k * 16, 16)] = lax.shift_right_logical(p, 16)

        unpack(0, c0, r0)
        pltpu.async_copy(h_hbm.at[c0], m0, sem)

        def pair_step(t, _):
            j0 = 2 * t
            j1 = j0 + 1
            unpack(j1, c1, r1)
            pltpu.make_async_copy(h_hbm.at[c0], m0, sem).wait()
            pltpu.async_copy(h_hbm.at[c1], m1, sem)
            pltpu.sync_copy(m0, acc_sh.at[r0], add=True)
            unpack(j0 + 2, c0, r0)
            pltpu.make_async_copy(h_hbm.at[c1], m1, sem).wait()
            pltpu.async_copy(h_hbm.at[c0], m0, sem)
            pltpu.sync_copy(m1, acc_sh.at[r1], add=True)
            return 0

        lax.fori_loop(0, (NCHUNKS - 1) // 2, pair_step, 0)
        pltpu.make_async_copy(h_hbm.at[c0], m0, sem).wait()
        pltpu.sync_copy(m0, acc_sh.at[r0], add=True)
        plsc.subcore_barrier()

        pltpu.sync_copy(acc_sh.at[pl.ds(base, ROWS_PER_TILE)],
                        acc_out.at[cid, pl.ds(base, ROWS_PER_TILE)])

    run = pl.kernel(body, out_type=out_type, mesh=mesh,
                    scratch_types=scratch)
    return run(h, packed3, zrows)


def _sc_degree(row3, ones, zrows):
    mesh = plsc.VectorSubcoreMesh(core_axis_name="c", subcore_axis_name="s")
    out_type = jax.ShapeDtypeStruct((NC, N_PAD, DEG_W), jnp.float32)
    scratch = [
        pltpu.VMEM((NCHUNKS, CHUNK), jnp.int32),
        pltpu.VMEM((CHUNK, DEG_W), jnp.float32),
        pltpu.VMEM_SHARED((N_PAD, DEG_W), jnp.float32),
    ]

    def body(row_hbm, ones_hbm, zd_hbm, deg_out, row_v, ones_v, deg_sh):
        cid = lax.axis_index("c")
        sid = lax.axis_index("s")
        wid = sid * NC + cid
        pltpu.sync_copy(row_hbm.at[wid], row_v)
        pltpu.sync_copy(ones_hbm, ones_v)
        base = sid * ROWS_PER_TILE
        pltpu.sync_copy(zd_hbm, deg_sh.at[pl.ds(base, ROWS_PER_TILE)])
        plsc.subcore_barrier()

        def chunk_step(j, _):
            pltpu.sync_copy(ones_v, deg_sh.at[row_v.at[j]], add=True)
            return 0

        lax.fori_loop(0, NCHUNKS, chunk_step, 0)
        plsc.subcore_barrier()
        pltpu.sync_copy(deg_sh.at[pl.ds(base, ROWS_PER_TILE)],
                        deg_out.at[cid, pl.ds(base, ROWS_PER_TILE)])

    run = pl.kernel(body, out_type=out_type, mesh=mesh,
                    scratch_types=scratch)
    return run(row3, ones, zrows)


BLK = 1000


def _inproj_body(x_ref, w_ref, b_ref, o_ref):
    o_ref[...] = (jnp.dot(x_ref[...], w_ref[...],
                          preferred_element_type=jnp.float32) + b_ref[...])


def _inproj(x, W, b):
    return pl.pallas_call(
        _inproj_body,
        grid=(N // BLK,),
        in_specs=[
            pl.BlockSpec((BLK, D), lambda i: (i, 0)),
            pl.BlockSpec((D, D), lambda i: (0, 0)),
            pl.BlockSpec((1, D), lambda i: (0, 0)),
        ],
        out_specs=pl.BlockSpec((BLK, D), lambda i: (i, 0)),
        out_shape=jax.ShapeDtypeStruct((N, D), jnp.float32),
    )(x, W, b.reshape(1, D))


def _combine_body(final, h_ref, acc_ref, deg_ref, wc_ref, bc_ref, g_ref,
                  be_ref, wo_ref, bo_ref, o_ref):
    agg = acc_ref[0] + acc_ref[1]
    deg = deg_ref[0, :, 0:1] + deg_ref[1, :, 0:1]
    agg = agg / jnp.maximum(deg, 1.0)
    hn = (jnp.dot(agg, wc_ref[...], preferred_element_type=jnp.float32)
          + bc_ref[...])
    h = h_ref[...] + hn
    mu = jnp.mean(h, axis=-1, keepdims=True)
    var = jnp.mean((h - mu) ** 2, axis=-1, keepdims=True)
    h = (h - mu) * lax.rsqrt(var + 1e-5) * g_ref[...] + be_ref[...]
    h = jnp.maximum(h, 0.0)
    if final:
        h = (jnp.dot(h, wo_ref[...], preferred_element_type=jnp.float32)
             + bo_ref[...])
    o_ref[...] = h


def _combine(h, acc, deg, Wc, bc, g, be, Wo, bo, final):
    return pl.pallas_call(
        functools.partial(_combine_body, final),
        grid=(N // BLK,),
        in_specs=[
            pl.BlockSpec((BLK, D), lambda i: (i, 0)),
            pl.BlockSpec((NC, BLK, D), lambda i: (0, i, 0)),
            pl.BlockSpec((NC, BLK, DEG_W), lambda i: (0, i, 0)),
            pl.BlockSpec((D, D), lambda i: (0, 0)),
            pl.BlockSpec((1, D), lambda i: (0, 0)),
            pl.BlockSpec((1, D), lambda i: (0, 0)),
            pl.BlockSpec((1, D), lambda i: (0, 0)),
            pl.BlockSpec((D, D), lambda i: (0, 0)),
            pl.BlockSpec((1, D), lambda i: (0, 0)),
        ],
        out_specs=pl.BlockSpec((BLK, D), lambda i: (i, 0)),
        out_shape=jax.ShapeDtypeStruct((N, D), jnp.float32),
    )(h, acc, deg, Wc, bc.reshape(1, D), g.reshape(1, D), be.reshape(1, D),
      Wo, bo.reshape(1, D))


def kernel(x, edge_index, W_in, b_in, W_c0, b_c0, W_c1, b_c1, g0, be0, g1,
           be1, W_out, b_out):
    pad = jnp.full((E_PAD - E,), N, dtype=jnp.int32)
    row_pad = jnp.concatenate([edge_index[0], pad])
    col_pad = jnp.concatenate([edge_index[1], jnp.zeros_like(pad)])
    row3 = row_pad.reshape(NW, NCHUNKS, CHUNK)
    packed3 = ((row_pad << 16) | col_pad).reshape(NW, NCHUNKS, CHUNK)
    zrows = jnp.zeros((ROWS_PER_TILE, D), jnp.float32)
    ones = jnp.ones((CHUNK, DEG_W), jnp.float32)

    h0 = _inproj(x, W_in, b_in)
    deg = _sc_degree(row3, ones, zrows)
    acc0 = _sc_aggregate(h0, packed3, zrows)
    h1 = _combine(h0, acc0, deg, W_c0, b_c0, g0, be0, W_out, b_out,
                  final=False)
    acc1 = _sc_aggregate(h1, packed3, zrows)
    out = _combine(h1, acc1, deg, W_c1, b_c1, g1, be1, W_out, b_out,
                   final=True)
    return out


# speedup vs baseline: 2.1706x; 2.1706x over previous
import functools

import jax
import jax.numpy as jnp
from jax import lax
from jax.experimental import pallas as pl
from jax.experimental.pallas import tpu as pltpu
from jax.experimental.pallas import tpu_sc as plsc

N = 10000
E = 320000
D = 128

NC = 2
NS = 16
NW = NC * NS
EDGES_PER_W = E // NW
CHUNK = 80
NCHUNKS = EDGES_PER_W // CHUNK
N_PAD = 10240
ROWS_PER_TILE = N_PAD // NS
DEG_W = 128


def _sc_aggregate(h, packed3, zrows):
    mesh = plsc.VectorSubcoreMesh(core_axis_name="c", subcore_axis_name="s")

    out_type = jax.ShapeDtypeStruct((NC, N_PAD, D), jnp.float32)

    scratch = [
        pltpu.VMEM((NCHUNKS, CHUNK), jnp.int32),
        pltpu.VMEM((CHUNK,), jnp.int32),
        pltpu.VMEM((CHUNK,), jnp.int32),
        pltpu.VMEM((CHUNK,), jnp.int32),
        pltpu.VMEM((CHUNK,), jnp.int32),
        pltpu.VMEM((CHUNK,), jnp.int32),
        pltpu.VMEM((CHUNK,), jnp.int32),
        pltpu.VMEM((CHUNK, D), jnp.float32),
        pltpu.VMEM((CHUNK, D), jnp.float32),
        pltpu.VMEM((CHUNK, D), jnp.float32),
        pltpu.VMEM_SHARED((N_PAD, D), jnp.float32),
        pltpu.SemaphoreType.DMA,
        pltpu.SemaphoreType.DMA,
    ]

    def body(h_hbm, packed_hbm, z_hbm, *rest):
        (acc_out, packed_v, c0, c1, c2, r0, r1, r2, m0, m1, m2, acc_sh,
         gsem, ssem) = rest
        cs = (c0, c1, c2)
        rs = (r0, r1, r2)
        ms = (m0, m1, m2)
        cid = lax.axis_index("c")
        sid = lax.axis_index("s")
        wid = sid * NC + cid

        pltpu.sync_copy(packed_hbm.at[wid], packed_v)

        base = sid * ROWS_PER_TILE
        pltpu.sync_copy(z_hbm, acc_sh.at[pl.ds(base, ROWS_PER_TILE)])
        plsc.subcore_barrier()

        def unpack(j, c_buf, r_buf):
            for k in range(CHUNK // 16):
                p = packed_v[j, pl.ds(k * 16, 16)]
                c_buf[pl.ds(k * 16, 16)] = jnp.bitwise_and(p, 0xFFFF)
                r_buf[pl.ds(k * 16, 16)] = lax.shift_right_logical(p, 16)

        def gather(j, b):
            pltpu.async_copy(h_hbm.at[cs[b]], ms[b], gsem)

        def gather_wait(b):
            pltpu.make_async_copy(h_hbm.at[cs[b]], ms[b], gsem).wait()

        def scatter(b):
            pltpu.async_copy(ms[b], acc_sh.at[rs[b]], ssem, add=True)

        def scatter_wait(b):
            pltpu.make_async_copy(ms[b], acc_sh.at[rs[b]], ssem).wait()

        unpack(0, c0, r0)
        gather(0, 0)
        unpack(1, c1, r1)
        gather(1, 1)
        gather_wait(0)
        scatter(0)
        unpack(2, c2, r2)
        gather(2, 2)

        def tri_step(t, _):
            for u in range(3):
                j = 3 * t + 1 + u
                b = (u + 1) % 3
                bp = u % 3
                gather_wait(b)
                scatter(b)
                scatter_wait(bp)
                jn = jnp.minimum(j + 2, NCHUNKS - 1)
                unpack(jn, cs[bp], rs[bp])
                gather(jn, bp)
            return 0

        lax.fori_loop(0, (NCHUNKS - 2) // 3, tri_step, 0)
        gather_wait(1)
        gather_wait(2)
        scatter(1)
        scatter_wait(0)
        scatter_wait(1)
        plsc.subcore_barrier()

        pltpu.sync_copy(acc_sh.at[pl.ds(base, ROWS_PER_TILE)],
                        acc_out.at[cid, pl.ds(base, ROWS_PER_TILE)])

    run = pl.kernel(body, out_type=out_type, mesh=mesh,
                    scratch_types=scratch)
    return run(h, packed3, zrows)


def _sc_degree(row3, ones, zrows):
    mesh = plsc.VectorSubcoreMesh(core_axis_name="c", subcore_axis_name="s")
    out_type = jax.ShapeDtypeStruct((NC, N_PAD, DEG_W), jnp.float32)
    scratch = [
        pltpu.VMEM((NCHUNKS, CHUNK), jnp.int32),
        pltpu.VMEM((CHUNK, DEG_W), jnp.float32),
        pltpu.VMEM_SHARED((N_PAD, DEG_W), jnp.float32),
        pltpu.SemaphoreType.DMA,
    ]

    def body(row_hbm, ones_hbm, zd_hbm, deg_out, row_v, ones_v, deg_sh,
             sem):
        cid = lax.axis_index("c")
        sid = lax.axis_index("s")
        wid = sid * NC + cid
        pltpu.sync_copy(row_hbm.at[wid], row_v)
        pltpu.sync_copy(ones_hbm, ones_v)
        base = sid * ROWS_PER_TILE
        pltpu.sync_copy(zd_hbm, deg_sh.at[pl.ds(base, ROWS_PER_TILE)])
        plsc.subcore_barrier()

        LAG = 4
        for j0 in range(LAG):
            pltpu.async_copy(ones_v, deg_sh.at[row_v.at[j0]], sem,
                             add=True)

        def chunk_step(j, _):
            pltpu.async_copy(ones_v, deg_sh.at[row_v.at[j + LAG]], sem,
                             add=True)
            pltpu.make_async_copy(ones_v, deg_sh.at[row_v.at[j]],
                                  sem).wait()
            return 0

        lax.fori_loop(0, NCHUNKS - LAG, chunk_step, 0)
        for j0 in range(LAG):
            pltpu.make_async_copy(ones_v, deg_sh.at[row_v.at[j0]],
                                  sem).wait()
        plsc.subcore_barrier()
        pltpu.sync_copy(deg_sh.at[pl.ds(base, ROWS_PER_TILE)],
                        deg_out.at[cid, pl.ds(base, ROWS_PER_TILE)])

    run = pl.kernel(body, out_type=out_type, mesh=mesh,
                    scratch_types=scratch)
    return run(row3, ones, zrows)


BLK = 1000


def _inproj_body(x_ref, w_ref, b_ref, o_ref):
    o_ref[...] = (jnp.dot(x_ref[...], w_ref[...],
                          preferred_element_type=jnp.float32) + b_ref[...])


def _inproj(x, W, b):
    return pl.pallas_call(
        _inproj_body,
        grid=(N // BLK,),
        in_specs=[
            pl.BlockSpec((BLK, D), lambda i: (i, 0)),
            pl.BlockSpec((D, D), lambda i: (0, 0)),
            pl.BlockSpec((1, D), lambda i: (0, 0)),
        ],
        out_specs=pl.BlockSpec((BLK, D), lambda i: (i, 0)),
        out_shape=jax.ShapeDtypeStruct((N, D), jnp.float32),
    )(x, W, b.reshape(1, D))


def _combine_body(final, h_ref, acc_ref, deg_ref, wc_ref, bc_ref, g_ref,
                  be_ref, wo_ref, bo_ref, o_ref):
    agg = acc_ref[0] + acc_ref[1]
    deg = deg_ref[0, :, 0:1] + deg_ref[1, :, 0:1]
    agg = agg / jnp.maximum(deg, 1.0)
    hn = (jnp.dot(agg, wc_ref[...], preferred_element_type=jnp.float32)
          + bc_ref[...])
    h = h_ref[...] + hn
    mu = jnp.mean(h, axis=-1, keepdims=True)
    var = jnp.mean((h - mu) ** 2, axis=-1, keepdims=True)
    h = (h - mu) * lax.rsqrt(var + 1e-5) * g_ref[...] + be_ref[...]
    h = jnp.maximum(h, 0.0)
    if final:
        h = (jnp.dot(h, wo_ref[...], preferred_element_type=jnp.float32)
             + bo_ref[...])
    o_ref[...] = h


def _combine(h, acc, deg, Wc, bc, g, be, Wo, bo, final):
    return pl.pallas_call(
        functools.partial(_combine_body, final),
        grid=(N // BLK,),
        in_specs=[
            pl.BlockSpec((BLK, D), lambda i: (i, 0)),
            pl.BlockSpec((NC, BLK, D), lambda i: (0, i, 0)),
            pl.BlockSpec((NC, BLK, DEG_W), lambda i: (0, i, 0)),
            pl.BlockSpec((D, D), lambda i: (0, 0)),
            pl.BlockSpec((1, D), lambda i: (0, 0)),
            pl.BlockSpec((1, D), lambda i: (0, 0)),
            pl.BlockSpec((1, D), lambda i: (0, 0)),
            pl.BlockSpec((D, D), lambda i: (0, 0)),
            pl.BlockSpec((1, D), lambda i: (0, 0)),
        ],
        out_specs=pl.BlockSpec((BLK, D), lambda i: (i, 0)),
        out_shape=jax.ShapeDtypeStruct((N, D), jnp.float32),
    )(h, acc, deg, Wc, bc.reshape(1, D), g.reshape(1, D), be.reshape(1, D),
      Wo, bo.reshape(1, D))


def kernel(x, edge_index, W_in, b_in, W_c0, b_c0, W_c1, b_c1, g0, be0, g1,
           be1, W_out, b_out):
    row3 = edge_index[0].reshape(NW, NCHUNKS, CHUNK)
    packed3 = ((edge_index[0] << 16) | edge_index[1]).reshape(
        NW, NCHUNKS, CHUNK)
    zrows = jnp.zeros((ROWS_PER_TILE, D), jnp.float32)
    ones = jnp.ones((CHUNK, DEG_W), jnp.float32)

    h0 = _inproj(x, W_in, b_in)
    deg = _sc_degree(row3, ones, zrows)
    acc0 = _sc_aggregate(h0, packed3, zrows)
    h1 = _combine(h0, acc0, deg, W_c0, b_c0, g0, be0, W_out, b_out,
                  final=False)
    acc1 = _sc_aggregate(h1, packed3, zrows)
    out = _combine(h1, acc1, deg, W_c1, b_c1, g1, be1, W_out, b_out,
                   final=True)
    return out


# speedup vs baseline: 2.2199x; 1.0227x over previous
import functools

import jax
import jax.numpy as jnp
from jax import lax
from jax.experimental import pallas as pl
from jax.experimental.pallas import tpu as pltpu
from jax.experimental.pallas import tpu_sc as plsc

N = 10000
E = 320000
D = 128

NC = 2
NS = 16
NW = NC * NS
EDGES_PER_W = E // NW
CHUNK = 80
NCHUNKS = EDGES_PER_W // CHUNK
N_PAD = 10240
ROWS_PER_TILE = N_PAD // NS
DEG_W = 128


def _sc_aggregate(h, packed3, zrows, ones=None):
    mesh = plsc.VectorSubcoreMesh(core_axis_name="c", subcore_axis_name="s")

    with_deg = ones is not None
    if with_deg:
        out_type = (jax.ShapeDtypeStruct((NC, N_PAD, D), jnp.float32),
                    jax.ShapeDtypeStruct((NC, N_PAD, D), jnp.float32))
    else:
        out_type = jax.ShapeDtypeStruct((NC, N_PAD, D), jnp.float32)

    scratch = [
        pltpu.VMEM((NCHUNKS, CHUNK), jnp.int32),
        pltpu.VMEM((CHUNK,), jnp.int32),
        pltpu.VMEM((CHUNK,), jnp.int32),
        pltpu.VMEM((CHUNK,), jnp.int32),
        pltpu.VMEM((CHUNK,), jnp.int32),
        pltpu.VMEM((CHUNK,), jnp.int32),
        pltpu.VMEM((CHUNK,), jnp.int32),
        pltpu.VMEM((CHUNK, D), jnp.float32),
        pltpu.VMEM((CHUNK, D), jnp.float32),
        pltpu.VMEM((CHUNK, D), jnp.float32),
        pltpu.VMEM_SHARED((N_PAD, D), jnp.float32),
        pltpu.SemaphoreType.DMA,
        pltpu.SemaphoreType.DMA,
    ]

    def body(h_hbm, packed_hbm, z_hbm, *rest):
        if with_deg:
            (ones_hbm, acc_out, deg_out, packed_v, c0, c1, c2, r0, r1,
             r2, m0, m1, m2, acc_sh, gsem, ssem) = rest
        else:
            (acc_out, packed_v, c0, c1, c2, r0, r1, r2, m0, m1, m2,
             acc_sh, gsem, ssem) = rest
        cs = (c0, c1, c2)
        rs = (r0, r1, r2)
        ms = (m0, m1, m2)
        cid = lax.axis_index("c")
        sid = lax.axis_index("s")
        wid = sid * NC + cid

        pltpu.sync_copy(packed_hbm.at[wid], packed_v)

        base = sid * ROWS_PER_TILE
        pltpu.sync_copy(z_hbm, acc_sh.at[pl.ds(base, ROWS_PER_TILE)])
        plsc.subcore_barrier()

        def unpack(j, c_buf, r_buf):
            for k in range(CHUNK // 16):
                p = packed_v[j, pl.ds(k * 16, 16)]
                c_buf[pl.ds(k * 16, 16)] = jnp.bitwise_and(p, 0xFFFF)
                r_buf[pl.ds(k * 16, 16)] = lax.shift_right_logical(p, 16)

        def gather(j, b):
            pltpu.async_copy(h_hbm.at[cs[b]], ms[b], gsem)

        def gather_wait(b):
            pltpu.make_async_copy(h_hbm.at[cs[b]], ms[b], gsem).wait()

        def scatter(b):
            pltpu.async_copy(ms[b], acc_sh.at[rs[b]], ssem, add=True)

        def scatter_wait(b):
            pltpu.make_async_copy(ms[b], acc_sh.at[rs[b]], ssem).wait()

        unpack(0, c0, r0)
        gather(0, 0)
        unpack(1, c1, r1)
        gather(1, 1)
        gather_wait(0)
        scatter(0)
        unpack(2, c2, r2)
        gather(2, 2)

        def tri_step(t, _):
            for u in range(3):
                j = 3 * t + 1 + u
                b = (u + 1) % 3
                bp = u % 3
                gather_wait(b)
                scatter(b)
                scatter_wait(bp)
                jn = jnp.minimum(j + 2, NCHUNKS - 1)
                unpack(jn, cs[bp], rs[bp])
                gather(jn, bp)
            return 0

        lax.fori_loop(0, (NCHUNKS - 2) // 3, tri_step, 0)
        gather_wait(1)
        gather_wait(2)
        scatter(1)
        scatter_wait(0)
        scatter_wait(1)
        plsc.subcore_barrier()

        pltpu.sync_copy(acc_sh.at[pl.ds(base, ROWS_PER_TILE)],
                        acc_out.at[cid, pl.ds(base, ROWS_PER_TILE)])

        if with_deg:
            pltpu.sync_copy(z_hbm, acc_sh.at[pl.ds(base, ROWS_PER_TILE)])
            pltpu.sync_copy(ones_hbm, m0)
            plsc.subcore_barrier()

            def unpack_rows(j, r_buf):
                for k in range(CHUNK // 16):
                    p = packed_v[j, pl.ds(k * 16, 16)]
                    r_buf[pl.ds(k * 16, 16)] = lax.shift_right_logical(
                        p, 16)

            def dscat(r_buf):
                pltpu.async_copy(m0, acc_sh.at[r_buf], ssem, add=True)

            def dwait(r_buf):
                pltpu.make_async_copy(m0, acc_sh.at[r_buf], ssem).wait()

            unpack_rows(0, r0)
            dscat(r0)
            unpack_rows(1, r1)
            dscat(r1)

            def deg_pair(t, _):
                j0 = 2 * t + 2
                dwait(r0)
                unpack_rows(j0, r0)
                dscat(r0)
                dwait(r1)
                unpack_rows(j0 + 1, r1)
                dscat(r1)
                return 0

            lax.fori_loop(0, (NCHUNKS - 3) // 2, deg_pair, 0)
            dwait(r0)
            unpack_rows(NCHUNKS - 1, r0)
            dscat(r0)
            dwait(r1)
            dwait(r0)
            plsc.subcore_barrier()
            pltpu.sync_copy(acc_sh.at[pl.ds(base, ROWS_PER_TILE)],
                            deg_out.at[cid, pl.ds(base, ROWS_PER_TILE)])

    run = pl.kernel(body, out_type=out_type, mesh=mesh,
                    scratch_types=scratch)
    if with_deg:
        return run(h, packed3, zrows, ones)
    return run(h, packed3, zrows)


BLK = 1000


def _inproj_body(x_ref, w_ref, b_ref, o_ref):
    o_ref[...] = (jnp.dot(x_ref[...], w_ref[...],
                          preferred_element_type=jnp.float32) + b_ref[...])


def _inproj(x, W, b):
    return pl.pallas_call(
        _inproj_body,
        grid=(N // BLK,),
        in_specs=[
            pl.BlockSpec((BLK, D), lambda i: (i, 0)),
            pl.BlockSpec((D, D), lambda i: (0, 0)),
            pl.BlockSpec((1, D), lambda i: (0, 0)),
        ],
        out_specs=pl.BlockSpec((BLK, D), lambda i: (i, 0)),
        out_shape=jax.ShapeDtypeStruct((N, D), jnp.float32),
    )(x, W, b.reshape(1, D))


def _combine_body(final, h_ref, acc_ref, deg_ref, wc_ref, bc_ref, g_ref,
                  be_ref, wo_ref, bo_ref, o_ref):
    agg = acc_ref[0] + acc_ref[1]
    deg = deg_ref[0, :, 0:1] + deg_ref[1, :, 0:1]
    agg = agg / jnp.maximum(deg, 1.0)
    hn = (jnp.dot(agg, wc_ref[...], preferred_element_type=jnp.float32)
          + bc_ref[...])
    h = h_ref[...] + hn
    mu = jnp.mean(h, axis=-1, keepdims=True)
    var = jnp.mean((h - mu) ** 2, axis=-1, keepdims=True)
    h = (h - mu) * lax.rsqrt(var + 1e-5) * g_ref[...] + be_ref[...]
    h = jnp.maximum(h, 0.0)
    if final:
        h = (jnp.dot(h, wo_ref[...], preferred_element_type=jnp.float32)
             + bo_ref[...])
    o_ref[...] = h


def _combine(h, acc, deg, Wc, bc, g, be, Wo, bo, final):
    return pl.pallas_call(
        functools.partial(_combine_body, final),
        grid=(N // BLK,),
        in_specs=[
            pl.BlockSpec((BLK, D), lambda i: (i, 0)),
            pl.BlockSpec((NC, BLK, D), lambda i: (0, i, 0)),
            pl.BlockSpec((NC, BLK, DEG_W), lambda i: (0, i, 0)),
            pl.BlockSpec((D, D), lambda i: (0, 0)),
            pl.BlockSpec((1, D), lambda i: (0, 0)),
            pl.BlockSpec((1, D), lambda i: (0, 0)),
            pl.BlockSpec((1, D), lambda i: (0, 0)),
            pl.BlockSpec((D, D), lambda i: (0, 0)),
            pl.BlockSpec((1, D), lambda i: (0, 0)),
        ],
        out_specs=pl.BlockSpec((BLK, D), lambda i: (i, 0)),
        out_shape=jax.ShapeDtypeStruct((N, D), jnp.float32),
    )(h, acc, deg, Wc, bc.reshape(1, D), g.reshape(1, D), be.reshape(1, D),
      Wo, bo.reshape(1, D))


def kernel(x, edge_index, W_in, b_in, W_c0, b_c0, W_c1, b_c1, g0, be0, g1,
           be1, W_out, b_out):
    packed3 = ((edge_index[0] << 16) | edge_index[1]).reshape(
        NW, NCHUNKS, CHUNK)
    zrows = jnp.zeros((ROWS_PER_TILE, D), jnp.float32)
    ones = jnp.ones((CHUNK, D), jnp.float32)

    h0 = _inproj(x, W_in, b_in)
    acc0, deg = _sc_aggregate(h0, packed3, zrows, ones)
    h1 = _combine(h0, acc0, deg, W_c0, b_c0, g0, be0, W_out, b_out,
                  final=False)
    acc1 = _sc_aggregate(h1, packed3, zrows)
    out = _combine(h1, acc1, deg, W_c1, b_c1, g1, be1, W_out, b_out,
                   final=True)
    return out


# speedup vs baseline: 2.2700x; 1.0225x over previous
import functools

import jax
import jax.numpy as jnp
from jax import lax
from jax.experimental import pallas as pl
from jax.experimental.pallas import tpu as pltpu
from jax.experimental.pallas import tpu_sc as plsc

N = 10000
E = 320000
D = 128

NC = 2
NS = 16
NW = NC * NS
EDGES_PER_W = E // NW
CHUNK = 80
NCHUNKS = EDGES_PER_W // CHUNK
N_PAD = 10240
ROWS_PER_TILE = N_PAD // NS
DEG_W = 128


def _sc_aggregate(h, packed3, zrows, ones=None):
    mesh = plsc.VectorSubcoreMesh(core_axis_name="c", subcore_axis_name="s")

    with_deg = ones is not None
    if with_deg:
        out_type = (jax.ShapeDtypeStruct((NC, N_PAD, D), jnp.float32),
                    jax.ShapeDtypeStruct((NC, N_PAD, D), jnp.float32))
    else:
        out_type = jax.ShapeDtypeStruct((NC, N_PAD, D), jnp.float32)

    scratch = [
        pltpu.VMEM((NCHUNKS, CHUNK), jnp.int32),
        pltpu.VMEM((CHUNK,), jnp.int32),
        pltpu.VMEM((CHUNK,), jnp.int32),
        pltpu.VMEM((CHUNK,), jnp.int32),
        pltpu.VMEM((CHUNK,), jnp.int32),
        pltpu.VMEM((CHUNK,), jnp.int32),
        pltpu.VMEM((CHUNK,), jnp.int32),
        pltpu.VMEM((CHUNK, D), jnp.float32),
        pltpu.VMEM((CHUNK, D), jnp.float32),
        pltpu.VMEM((CHUNK, D), jnp.float32),
        pltpu.VMEM_SHARED((N_PAD, D), jnp.float32),
        pltpu.SemaphoreType.DMA,
        pltpu.SemaphoreType.DMA,
    ]

    def body(h_hbm, packed_hbm, z_hbm, *rest):
        if with_deg:
            (ones_hbm, acc_out, deg_out, packed_v, c0, c1, c2, r0, r1,
             r2, m0, m1, m2, acc_sh, gsem, ssem) = rest
        else:
            (acc_out, packed_v, c0, c1, c2, r0, r1, r2, m0, m1, m2,
             acc_sh, gsem, ssem) = rest
        cs = (c0, c1, c2)
        rs = (r0, r1, r2)
        ms = (m0, m1, m2)
        cid = lax.axis_index("c")
        sid = lax.axis_index("s")
        wid = sid * NC + cid

        pltpu.sync_copy(packed_hbm.at[wid], packed_v)

        base = sid * ROWS_PER_TILE
        pltpu.sync_copy(z_hbm, acc_sh.at[pl.ds(base, ROWS_PER_TILE)])
        plsc.subcore_barrier()

        def unpack(j, c_buf, r_buf):
            for k in range(CHUNK // 16):
                p = packed_v[j, pl.ds(k * 16, 16)]
                c_buf[pl.ds(k * 16, 16)] = jnp.bitwise_and(p, 0xFFFF)
                r_buf[pl.ds(k * 16, 16)] = lax.shift_right_logical(p, 16)

        def gather(j, b):
            pltpu.async_copy(h_hbm.at[cs[b]], ms[b], gsem)

        def gather_wait(b):
            pltpu.make_async_copy(h_hbm.at[cs[b]], ms[b], gsem).wait()

        def scatter(b):
            pltpu.async_copy(ms[b], acc_sh.at[rs[b]], ssem, add=True)

        def scatter_wait(b):
            pltpu.make_async_copy(ms[b], acc_sh.at[rs[b]], ssem).wait()

        unpack(0, c0, r0)
        gather(0, 0)
        unpack(1, c1, r1)
        gather(1, 1)
        gather_wait(0)
        scatter(0)
        unpack(2, c2, r2)
        gather(2, 2)

        def tri_step(t, _):
            for u in range(3):
                j = 3 * t + 1 + u
                b = (u + 1) % 3
                bp = u % 3
                gather_wait(b)
                scatter(b)
                scatter_wait(bp)
                jn = jnp.minimum(j + 2, NCHUNKS - 1)
                unpack(jn, cs[bp], rs[bp])
                gather(jn, bp)
            return 0

        lax.fori_loop(0, (NCHUNKS - 2) // 3, tri_step, 0)
        gather_wait(1)
        gather_wait(2)
        scatter(1)
        scatter_wait(0)
        scatter_wait(1)
        plsc.subcore_barrier()

        pltpu.sync_copy(acc_sh.at[pl.ds(base, ROWS_PER_TILE)],
                        acc_out.at[cid, pl.ds(base, ROWS_PER_TILE)])

        if with_deg:
            pltpu.sync_copy(z_hbm, acc_sh.at[pl.ds(base, ROWS_PER_TILE)])
            pltpu.sync_copy(ones_hbm, m0)
            plsc.subcore_barrier()

            def unpack_rows(j, r_buf):
                for k in range(CHUNK // 16):
                    p = packed_v[j, pl.ds(k * 16, 16)]
                    r_buf[pl.ds(k * 16, 16)] = lax.shift_right_logical(
                        p, 16)

            def dscat(r_buf):
                pltpu.async_copy(m0, acc_sh.at[r_buf], ssem, add=True)

            def dwait(r_buf):
                pltpu.make_async_copy(m0, acc_sh.at[r_buf], ssem).wait()

            unpack_rows(0, r0)
            dscat(r0)
            unpack_rows(1, r1)
            dscat(r1)

            def deg_pair(t, _):
                j0 = 2 * t + 2
                dwait(r0)
                unpack_rows(j0, r0)
                dscat(r0)
                dwait(r1)
                unpack_rows(j0 + 1, r1)
                dscat(r1)
                return 0

            lax.fori_loop(0, (NCHUNKS - 3) // 2, deg_pair, 0)
            dwait(r0)
            unpack_rows(NCHUNKS - 1, r0)
            dscat(r0)
            dwait(r1)
            dwait(r0)
            plsc.subcore_barrier()
            pltpu.sync_copy(acc_sh.at[pl.ds(base, ROWS_PER_TILE)],
                            deg_out.at[cid, pl.ds(base, ROWS_PER_TILE)])

    run = pl.kernel(body, out_type=out_type, mesh=mesh,
                    scratch_types=scratch)
    if with_deg:
        return run(h, packed3, zrows, ones)
    return run(h, packed3, zrows)


BLK = 2000


def _inproj_body(x_ref, w_ref, b_ref, o_ref):
    o_ref[...] = (jnp.dot(x_ref[...], w_ref[...],
                          preferred_element_type=jnp.float32) + b_ref[...])


def _inproj(x, W, b):
    return pl.pallas_call(
        _inproj_body,
        grid=(N // BLK,),
        in_specs=[
            pl.BlockSpec((BLK, D), lambda i: (i, 0)),
            pl.BlockSpec((D, D), lambda i: (0, 0)),
            pl.BlockSpec((1, D), lambda i: (0, 0)),
        ],
        out_specs=pl.BlockSpec((BLK, D), lambda i: (i, 0)),
        out_shape=jax.ShapeDtypeStruct((N, D), jnp.float32),
    )(x, W, b.reshape(1, D))


def _combine_body(final, h_ref, acc_ref, deg_ref, wc_ref, bc_ref, g_ref,
                  be_ref, wo_ref, bo_ref, o_ref):
    agg = acc_ref[0] + acc_ref[1]
    deg = deg_ref[0, :, 0:1] + deg_ref[1, :, 0:1]
    agg = agg / jnp.maximum(deg, 1.0)
    hn = (jnp.dot(agg, wc_ref[...], preferred_element_type=jnp.float32)
          + bc_ref[...])
    h = h_ref[...] + hn
    mu = jnp.mean(h, axis=-1, keepdims=True)
    var = jnp.mean((h - mu) ** 2, axis=-1, keepdims=True)
    h = (h - mu) * lax.rsqrt(var + 1e-5) * g_ref[...] + be_ref[...]
    h = jnp.maximum(h, 0.0)
    if final:
        h = (jnp.dot(h, wo_ref[...], preferred_element_type=jnp.float32)
             + bo_ref[...])
    o_ref[...] = h


def _combine(h, acc, deg, Wc, bc, g, be, Wo, bo, final):
    return pl.pallas_call(
        functools.partial(_combine_body, final),
        grid=(N // BLK,),
        in_specs=[
            pl.BlockSpec((BLK, D), lambda i: (i, 0)),
            pl.BlockSpec((NC, BLK, D), lambda i: (0, i, 0)),
            pl.BlockSpec((NC, BLK, DEG_W), lambda i: (0, i, 0)),
            pl.BlockSpec((D, D), lambda i: (0, 0)),
            pl.BlockSpec((1, D), lambda i: (0, 0)),
            pl.BlockSpec((1, D), lambda i: (0, 0)),
            pl.BlockSpec((1, D), lambda i: (0, 0)),
            pl.BlockSpec((D, D), lambda i: (0, 0)),
            pl.BlockSpec((1, D), lambda i: (0, 0)),
        ],
        out_specs=pl.BlockSpec((BLK, D), lambda i: (i, 0)),
        out_shape=jax.ShapeDtypeStruct((N, D), jnp.float32),
    )(h, acc, deg, Wc, bc.reshape(1, D), g.reshape(1, D), be.reshape(1, D),
      Wo, bo.reshape(1, D))


def kernel(x, edge_index, W_in, b_in, W_c0, b_c0, W_c1, b_c1, g0, be0, g1,
           be1, W_out, b_out):
    packed3 = ((edge_index[0] << 16) | edge_index[1]).reshape(
        NW, NCHUNKS, CHUNK)
    zrows = jnp.zeros((ROWS_PER_TILE, D), jnp.float32)
    ones = jnp.ones((CHUNK, D), jnp.float32)

    h0 = _inproj(x, W_in, b_in)
    acc0, deg = _sc_aggregate(h0, packed3, zrows, ones)
    h1 = _combine(h0, acc0, deg, W_c0, b_c0, g0, be0, W_out, b_out,
                  final=False)
    acc1 = _sc_aggregate(h1, packed3, zrows)
    out = _combine(h1, acc1, deg, W_c1, b_c1, g1, be1, W_out, b_out,
                   final=True)
    return out


# speedup vs baseline: 2.2959x; 1.0114x over previous
import functools

import jax
import jax.numpy as jnp
from jax import lax
from jax.experimental import pallas as pl
from jax.experimental.pallas import tpu as pltpu
from jax.experimental.pallas import tpu_sc as plsc

N = 10000
E = 320000
D = 128

NC = 2
NS = 16
NW = NC * NS
EDGES_PER_W = E // NW
CHUNK = 80
NCHUNKS = EDGES_PER_W // CHUNK
N_PAD = 10240
ROWS_PER_TILE = N_PAD // NS
DEG_W = 128


def _sc_aggregate(h, packed3, zrows, ones=None):
    mesh = plsc.VectorSubcoreMesh(core_axis_name="c", subcore_axis_name="s")

    with_deg = ones is not None
    if with_deg:
        out_type = (jax.ShapeDtypeStruct((NC, N_PAD, D), jnp.float32),
                    jax.ShapeDtypeStruct((NC, N_PAD, D), jnp.float32))
    else:
        out_type = jax.ShapeDtypeStruct((NC, N_PAD, D), jnp.float32)

    scratch = [
        pltpu.VMEM((NCHUNKS, CHUNK), jnp.int32),
        pltpu.VMEM((CHUNK,), jnp.int32),
        pltpu.VMEM((CHUNK,), jnp.int32),
        pltpu.VMEM((CHUNK,), jnp.int32),
        pltpu.VMEM((CHUNK,), jnp.int32),
        pltpu.VMEM((CHUNK,), jnp.int32),
        pltpu.VMEM((CHUNK,), jnp.int32),
        pltpu.VMEM((CHUNK, D), jnp.float32),
        pltpu.VMEM((CHUNK, D), jnp.float32),
        pltpu.VMEM((CHUNK, D), jnp.float32),
        pltpu.VMEM_SHARED((N_PAD, D), jnp.float32),
        pltpu.SemaphoreType.DMA,
        pltpu.SemaphoreType.DMA,
    ]

    def body(h_hbm, packed_hbm, z_hbm, *rest):
        if with_deg:
            (ones_hbm, acc_out, deg_out, packed_v, c0, c1, c2, r0, r1,
             r2, m0, m1, m2, acc_sh, gsem, ssem) = rest
        else:
            (acc_out, packed_v, c0, c1, c2, r0, r1, r2, m0, m1, m2,
             acc_sh, gsem, ssem) = rest
        cs = (c0, c1, c2)
        rs = (r0, r1, r2)
        ms = (m0, m1, m2)
        cid = lax.axis_index("c")
        sid = lax.axis_index("s")
        wid = sid * NC + cid

        pltpu.sync_copy(packed_hbm.at[wid], packed_v)

        base = sid * ROWS_PER_TILE
        pltpu.sync_copy(z_hbm, acc_sh.at[pl.ds(base, ROWS_PER_TILE)])
        plsc.subcore_barrier()

        def unpack(j, c_buf, r_buf):
            for k in range(CHUNK // 16):
                p = packed_v[j, pl.ds(k * 16, 16)]
                c_buf[pl.ds(k * 16, 16)] = jnp.bitwise_and(p, 0xFFFF)
                r_buf[pl.ds(k * 16, 16)] = lax.shift_right_logical(p, 16)

        def gather(j, b):
            pltpu.async_copy(h_hbm.at[cs[b]], ms[b], gsem)

        def gather_wait(b):
            pltpu.make_async_copy(h_hbm.at[cs[b]], ms[b], gsem).wait()

        def scatter(b):
            pltpu.async_copy(ms[b], acc_sh.at[rs[b]], ssem, add=True)

        def scatter_wait(b):
            pltpu.make_async_copy(ms[b], acc_sh.at[rs[b]], ssem).wait()

        unpack(0, c0, r0)
        gather(0, 0)
        unpack(1, c1, r1)
        gather(1, 1)
        gather_wait(0)
        scatter(0)
        unpack(2, c2, r2)
        gather(2, 2)

        def tri_step(t, _):
            for u in range(3):
                j = 3 * t + 1 + u
                b = (u + 1) % 3
                bp = u % 3
                gather_wait(b)
                scatter(b)
                scatter_wait(bp)
                jn = jnp.minimum(j + 2, NCHUNKS - 1)
                unpack(jn, cs[bp], rs[bp])
                gather(jn, bp)
            return 0

        lax.fori_loop(0, (NCHUNKS - 2) // 3, tri_step, 0)
        gather_wait(1)
        gather_wait(2)
        scatter(1)
        scatter_wait(0)
        scatter_wait(1)
        plsc.subcore_barrier()

        pltpu.sync_copy(acc_sh.at[pl.ds(base, ROWS_PER_TILE)],
                        acc_out.at[cid, pl.ds(base, ROWS_PER_TILE)])

        if with_deg:
            pltpu.sync_copy(z_hbm, acc_sh.at[pl.ds(base, ROWS_PER_TILE)])
            pltpu.sync_copy(ones_hbm, m0)
            plsc.subcore_barrier()

            def unpack_rows(j, r_buf):
                for k in range(CHUNK // 16):
                    p = packed_v[j, pl.ds(k * 16, 16)]
                    r_buf[pl.ds(k * 16, 16)] = lax.shift_right_logical(
                        p, 16)

            def dscat(r_buf):
                pltpu.async_copy(m0, acc_sh.at[r_buf], ssem, add=True)

            def dwait(r_buf):
                pltpu.make_async_copy(m0, acc_sh.at[r_buf], ssem).wait()

            unpack_rows(0, r0)
            dscat(r0)
            unpack_rows(1, r1)
            dscat(r1)

            def deg_pair(t, _):
                j0 = 2 * t + 2
                dwait(r0)
                unpack_rows(j0, r0)
                dscat(r0)
                dwait(r1)
                unpack_rows(j0 + 1, r1)
                dscat(r1)
                return 0

            lax.fori_loop(0, (NCHUNKS - 3) // 2, deg_pair, 0)
            dwait(r0)
            unpack_rows(NCHUNKS - 1, r0)
            dscat(r0)
            dwait(r1)
            dwait(r0)
            plsc.subcore_barrier()
            pltpu.sync_copy(acc_sh.at[pl.ds(base, ROWS_PER_TILE)],
                            deg_out.at[cid, pl.ds(base, ROWS_PER_TILE)])

    run = pl.kernel(body, out_type=out_type, mesh=mesh,
                    scratch_types=scratch)
    if with_deg:
        return run(h, packed3, zrows, ones)
    return run(h, packed3, zrows)


BLK = 5000


def _inproj_body(x_ref, w_ref, b_ref, o_ref):
    o_ref[...] = (jnp.dot(x_ref[...], w_ref[...],
                          preferred_element_type=jnp.float32) + b_ref[...])


def _inproj(x, W, b):
    return pl.pallas_call(
        _inproj_body,
        grid=(N // BLK,),
        in_specs=[
            pl.BlockSpec((BLK, D), lambda i: (i, 0)),
            pl.BlockSpec((D, D), lambda i: (0, 0)),
            pl.BlockSpec((1, D), lambda i: (0, 0)),
        ],
        out_specs=pl.BlockSpec((BLK, D), lambda i: (i, 0)),
        out_shape=jax.ShapeDtypeStruct((N, D), jnp.float32),
    )(x, W, b.reshape(1, D))


def _combine_body(final, h_ref, acc_ref, deg_ref, wc_ref, bc_ref, g_ref,
                  be_ref, wo_ref, bo_ref, o_ref):
    agg = acc_ref[0] + acc_ref[1]
    deg = deg_ref[0, :, 0:1] + deg_ref[1, :, 0:1]
    agg = agg / jnp.maximum(deg, 1.0)
    hn = (jnp.dot(agg, wc_ref[...], preferred_element_type=jnp.float32)
          + bc_ref[...])
    h = h_ref[...] + hn
    mu = jnp.mean(h, axis=-1, keepdims=True)
    var = jnp.mean((h - mu) ** 2, axis=-1, keepdims=True)
    h = (h - mu) * lax.rsqrt(var + 1e-5) * g_ref[...] + be_ref[...]
    h = jnp.maximum(h, 0.0)
    if final:
        h = (jnp.dot(h, wo_ref[...], preferred_element_type=jnp.float32)
             + bo_ref[...])
    o_ref[...] = h


def _combine(h, acc, deg, Wc, bc, g, be, Wo, bo, final):
    return pl.pallas_call(
        functools.partial(_combine_body, final),
        grid=(N // BLK,),
        in_specs=[
            pl.BlockSpec((BLK, D), lambda i: (i, 0)),
            pl.BlockSpec((NC, BLK, D), lambda i: (0, i, 0)),
            pl.BlockSpec((NC, BLK, DEG_W), lambda i: (0, i, 0)),
            pl.BlockSpec((D, D), lambda i: (0, 0)),
            pl.BlockSpec((1, D), lambda i: (0, 0)),
            pl.BlockSpec((1, D), lambda i: (0, 0)),
            pl.BlockSpec((1, D), lambda i: (0, 0)),
            pl.BlockSpec((D, D), lambda i: (0, 0)),
            pl.BlockSpec((1, D), lambda i: (0, 0)),
        ],
        out_specs=pl.BlockSpec((BLK, D), lambda i: (i, 0)),
        out_shape=jax.ShapeDtypeStruct((N, D), jnp.float32),
    )(h, acc, deg, Wc, bc.reshape(1, D), g.reshape(1, D), be.reshape(1, D),
      Wo, bo.reshape(1, D))


def kernel(x, edge_index, W_in, b_in, W_c0, b_c0, W_c1, b_c1, g0, be0, g1,
           be1, W_out, b_out):
    packed3 = ((edge_index[0] << 16) | edge_index[1]).reshape(
        NW, NCHUNKS, CHUNK)
    zrows = jnp.zeros((ROWS_PER_TILE, D), jnp.float32)
    ones = jnp.ones((CHUNK, D), jnp.float32)

    h0 = _inproj(x, W_in, b_in)
    acc0, deg = _sc_aggregate(h0, packed3, zrows, ones)
    h1 = _combine(h0, acc0, deg, W_c0, b_c0, g0, be0, W_out, b_out,
                  final=False)
    acc1 = _sc_aggregate(h1, packed3, zrows)
    out = _combine(h1, acc1, deg, W_c1, b_c1, g1, be1, W_out, b_out,
                   final=True)
    return out
